# Initial kernel scaffold; baseline (speedup 1.0000x reference)
import jax
import jax.numpy as jnp
from jax.experimental import pallas as pl


def kernel(x, edge_index, W1, b1, gamma1, beta1, W2, b2, gamma2, beta2):
    raise NotImplementedError("write your pallas kernel here")


# speedup vs baseline: 19.5724x; 19.5724x over previous
import functools

import jax
import jax.numpy as jnp
from jax import lax
from jax.experimental import pallas as pl
from jax.experimental.pallas import tpu as pltpu
from jax.experimental.pallas import tpu_sc as plsc

N = 10000
E = 320000
D_IN = 128
D_H = 64

NC = 2
NS = 16
NW = NC * NS
CH = 128
K = 79
EPW = K * CH
EPAD = NW * EPW
N2 = 10240
RPT = N // NS
ZPT = N2 // NS

_mesh = plsc.VectorSubcoreMesh(core_axis_name="c", subcore_axis_name="s")
_sc_params = pltpu.CompilerParams(use_tc_tiling_on_sc=False)


@functools.partial(
    pl.kernel,
    out_type=jax.ShapeDtypeStruct((NC, N2), jnp.float32),
    mesh=_mesh,
    compiler_params=_sc_params,
    scratch_types=[
        pltpu.VMEM((K, CH), jnp.int32),
        pltpu.VMEM((CH,), jnp.float32),
        pltpu.VMEM((ZPT,), jnp.float32),
        pltpu.VMEM_SHARED((N2,), jnp.float32),
        pltpu.SemaphoreType.DMA,
    ],
)
def _deg_kernel(dst_hbm, ones_hbm, zz_hbm, out_hbm, dstv, onesv, zbuf, hist, sem):
    c = lax.axis_index("c")
    s = lax.axis_index("s")
    wid = s * NC + c
    pltpu.sync_copy(dst_hbm.at[wid], dstv)
    pltpu.sync_copy(ones_hbm, onesv)
    pltpu.sync_copy(zz_hbm, zbuf)
    pltpu.sync_copy(zbuf, hist.at[pl.ds(s * ZPT, ZPT)])
    plsc.subcore_barrier()

    def body(k, carry):
        pltpu.async_copy(onesv, hist.at[dstv.at[k]], sem, add=True).wait()
        return carry

    lax.fori_loop(0, K, body, 0)
    plsc.subcore_barrier()
    pltpu.sync_copy(hist.at[pl.ds(s * ZPT, ZPT)], out_hbm.at[c, pl.ds(s * ZPT, ZPT)])


@functools.partial(
    pl.kernel,
    out_type=jax.ShapeDtypeStruct((NC, N2, D_H), jnp.float32),
    mesh=_mesh,
    compiler_params=_sc_params,
    scratch_types=[
        pltpu.VMEM((K, CH), jnp.int32),
        pltpu.VMEM((K, CH), jnp.int32),
        pltpu.VMEM((CH, D_H), jnp.float32),
        pltpu.VMEM_SHARED((N2, D_H), jnp.float32),
        pltpu.SemaphoreType.DMA,
        pltpu.SemaphoreType.DMA,
    ],
)
def _edge_kernel(g_hbm, src_hbm, dst_hbm, zz_hbm, out_hbm,
                 srcv, dstv, rows, acc, gsem, ssem):
    c = lax.axis_index("c")
    s = lax.axis_index("s")
    wid = s * NC + c
    pltpu.sync_copy(src_hbm.at[wid], srcv)
    pltpu.sync_copy(dst_hbm.at[wid], dstv)
    pltpu.sync_copy(zz_hbm, rows)
    for j in range(ZPT // CH):
        pltpu.sync_copy(rows, acc.at[pl.ds(s * ZPT + j * CH, CH)])
    plsc.subcore_barrier()

    def body(k, carry):
        pltpu.async_copy(g_hbm.at[srcv.at[k]], rows, gsem).wait()
        pltpu.async_copy(rows, acc.at[dstv.at[k]], ssem, add=True).wait()
        return carry

    lax.fori_loop(0, K, body, 0)
    plsc.subcore_barrier()
    pltpu.sync_copy(acc.at[pl.ds(s * ZPT, ZPT)],
                    out_hbm.at[c, pl.ds(s * ZPT, ZPT)])


def _prep_body(x_ref, w1_ref, degp_ref, g_ref, dinv_ref):
    deg = degp_ref[0] + degp_ref[1] + 1.0
    dinv = lax.rsqrt(deg)
    h = jnp.dot(x_ref[...], w1_ref[...], preferred_element_type=jnp.float32)
    g_ref[...] = h * dinv
    dinv_ref[...] = dinv


def _mid_body(part_ref, g1_ref, dinv_ref, b1_ref, gm1_ref, bt1_ref, w2_ref,
              g2_ref):
    ssum = part_ref[0, :N, :] + part_ref[1, :N, :] + g1_ref[...]
    out1 = ssum * dinv_ref[...] + b1_ref[...]
    mu = jnp.mean(out1, axis=0, keepdims=True)
    var = jnp.mean((out1 - mu) ** 2, axis=0, keepdims=True)
    hbn = (out1 - mu) * lax.rsqrt(var + 1e-5) * gm1_ref[...] + bt1_ref[...]
    hrelu = jnp.maximum(hbn, 0.0)
    h2 = jnp.dot(hrelu, w2_ref[...], preferred_element_type=jnp.float32)
    g2_ref[...] = h2 * dinv_ref[...]


def _fin_body(part_ref, g2_ref, dinv_ref, b2_ref, gm2_ref, bt2_ref, out_ref):
    ssum = part_ref[0, :N, :] + part_ref[1, :N, :] + g2_ref[...]
    out2 = ssum * dinv_ref[...] + b2_ref[...]
    mu = jnp.mean(out2, axis=0, keepdims=True)
    var = jnp.mean((out2 - mu) ** 2, axis=0, keepdims=True)
    out_ref[...] = (out2 - mu) * lax.rsqrt(var + 1e-5) * gm2_ref[...] + bt2_ref[...]


_prep_call = pl.pallas_call(
    _prep_body,
    out_shape=[jax.ShapeDtypeStruct((N, D_H), jnp.float32),
               jax.ShapeDtypeStruct((N, 1), jnp.float32)],
)

_mid_call = pl.pallas_call(
    _mid_body,
    out_shape=jax.ShapeDtypeStruct((N, D_H), jnp.float32),
)

_fin_call = pl.pallas_call(
    _fin_body,
    out_shape=jax.ShapeDtypeStruct((N, D_H), jnp.float32),
)


def kernel(x, edge_index, W1, b1, gamma1, beta1, W2, b2, gamma2, beta2):
    pad = EPAD - E
    src = jnp.concatenate([edge_index[0], jnp.zeros((pad,), jnp.int32)])
    dst = jnp.concatenate([edge_index[1], jnp.full((pad,), N, jnp.int32)])
    srcp = src.reshape(NW, K, CH)
    dstp = dst.reshape(NW, K, CH)

    ones_c = jnp.ones((CH,), jnp.float32)
    zz1 = jnp.zeros((ZPT,), jnp.float32)
    zz2 = jnp.zeros((CH, D_H), jnp.float32)

    degp = _deg_kernel(dstp, ones_c, zz1)
    degp2 = degp[:, :N, None]

    g1, dinv = _prep_call(x, W1, degp2)

    part1 = _edge_kernel(g1, srcp, dstp, zz2)
    g2 = _mid_call(part1, g1, dinv, b1[None, :], gamma1[None, :],
                   beta1[None, :], W2)

    part2 = _edge_kernel(g2, srcp, dstp, zz2)
    out = _fin_call(part2, g2, dinv, b2[None, :], gamma2[None, :],
                    beta2[None, :])
    return out
